# Initial kernel scaffold; baseline (speedup 1.0000x reference)
import jax
import jax.numpy as jnp
from jax.experimental import pallas as pl


def kernel(features, edge_index, edge_weight, W, b):
    raise NotImplementedError("write your pallas kernel here")


# speedup vs baseline: 5.4914x; 5.4914x over previous
import functools

import jax
import jax.numpy as jnp
from jax import lax
from jax.experimental import pallas as pl
from jax.experimental.pallas import tpu as pltpu
from jax.experimental.pallas import tpu_sc as plsc

N_NODES = 10000
N_EDGES = 320000
D = 128
CHUNK = 128
N_CHUNKS = N_EDGES // CHUNK
NW = 32
WB = 40
N_WB = N_NODES // WB


def _sc_aggregate(features, src, dst, w):
    mesh = plsc.VectorSubcoreMesh(core_axis_name="c", subcore_axis_name="s")

    @functools.partial(
        pl.kernel,
        mesh=mesh,
        out_type=jax.ShapeDtypeStruct((2, N_NODES, D), jnp.float32),
        scratch_types=[
            pltpu.VMEM((CHUNK,), jnp.int32),
            pltpu.VMEM((CHUNK,), jnp.int32),
            pltpu.VMEM((CHUNK,), jnp.float32),
            pltpu.VMEM((CHUNK, D), jnp.float32),
            pltpu.VMEM((WB, D), jnp.float32),
            pltpu.VMEM_SHARED((N_NODES, D), jnp.float32),
            pltpu.SemaphoreType.DMA,
        ],
    )
    def agg(feat_hbm, src_hbm, dst_hbm, w_hbm, out_hbm,
            srcbuf, dstbuf, wbuf, rows, bounce, acc, sem):
        c = lax.axis_index("c")
        s = lax.axis_index("s")
        wid = s * 2 + c

        zero = jnp.zeros((16,), jnp.float32)

        def zrow(r, _):
            for j in range(D // 16):
                bounce[r, pl.ds(j * 16, 16)] = zero
            return 0

        lax.fori_loop(0, WB, zrow, 0)

        for k in range(N_WB // 16 + 1):
            blk = s + k * 16

            @pl.when(blk < N_WB)
            def _():
                pltpu.sync_copy(bounce, acc.at[pl.ds(blk * WB, WB)])

        plsc.subcore_barrier()

        n_even = N_CHUNKS // NW
        nloc = n_even + jnp.where(wid < N_CHUNKS - n_even * NW, 1, 0)

        def chunk_body(k, _):
            base = (wid + k * NW) * CHUNK
            pltpu.sync_copy(src_hbm.at[pl.ds(base, CHUNK)], srcbuf)
            pltpu.sync_copy(dst_hbm.at[pl.ds(base, CHUNK)], dstbuf)
            pltpu.sync_copy(w_hbm.at[pl.ds(base, CHUNK)], wbuf)
            pltpu.async_copy(feat_hbm.at[srcbuf], rows, sem).wait()

            def scale_16(k16, _):
                wv = wbuf[pl.ds(k16 * 16, 16)]
                for i in range(16):
                    wi = wv[i]
                    e = k16 * 16 + i
                    for j in range(D // 16):
                        sl = pl.ds(j * 16, 16)
                        rows[e, sl] = rows[e, sl] * wi
                return 0

            lax.fori_loop(0, CHUNK // 16, scale_16, 0)
            pltpu.sync_copy(rows, acc.at[dstbuf], add=True)
            return 0

        lax.fori_loop(0, nloc, chunk_body, 0)
        plsc.subcore_barrier()

        for k in range(N_WB // 16 + 1):
            blk = s + k * 16

            @pl.when(blk < N_WB)
            def _():
                pltpu.sync_copy(acc.at[pl.ds(blk * WB, WB)], bounce)
                pltpu.sync_copy(bounce, out_hbm.at[c, pl.ds(blk * WB, WB)])

    return agg(features, src, dst, w)


def _tc_linear_body(p_ref, f_ref, w_ref, b_ref, o_ref):
    x = p_ref[0] + p_ref[1] + f_ref[...]
    o_ref[...] = lax.dot_general(
        x, w_ref[...], (((1,), (1,)), ((), ())),
        preferred_element_type=jnp.float32) + b_ref[...]


def kernel(features, edge_index, edge_weight, W, b):
    src = edge_index[0].astype(jnp.int32)
    dst = edge_index[1].astype(jnp.int32)
    w = edge_weight.astype(jnp.float32)

    partials = _sc_aggregate(features, src, dst, w)

    blk = 2000
    out = pl.pallas_call(
        _tc_linear_body,
        grid=(N_NODES // blk,),
        in_specs=[
            pl.BlockSpec((2, blk, D), lambda i: (0, i, 0)),
            pl.BlockSpec((blk, D), lambda i: (i, 0)),
            pl.BlockSpec((D, D), lambda i: (0, 0)),
            pl.BlockSpec((1, D), lambda i: (0, 0)),
        ],
        out_specs=pl.BlockSpec((blk, D), lambda i: (i, 0)),
        out_shape=jax.ShapeDtypeStruct((N_NODES, D), jnp.float32),
    )(partials, features, W, b.reshape(1, D))
    return out
